# baseline (speedup 1.0000x reference)
import functools

import jax
import jax.numpy as jnp
from jax import lax
from jax.experimental import pallas as pl
from jax.experimental.pallas import tpu as pltpu
from jax.experimental.pallas import tpu_sc as plsc

N = 10000
E = 320000
F = 128
NENV = 4
NP = 10240
NSC = 2
NTILE = 16
TPE = E // (NSC * NTILE)
CH = 128
NCHUNK = 79
PADE = NCHUNK * CH - TPE
RPT = NP // NTILE

_MESH = plsc.VectorSubcoreMesh(core_axis_name="c", subcore_axis_name="s")


def _zero_vmem_2d(ref, nrows, ncols):
    z = jnp.zeros((16,), jnp.float32)
    cpr = ncols // 16

    def body(i, _):
        r = i // cpr
        c = i % cpr
        ref[r, pl.ds(c * 16, 16)] = z
        return 0

    lax.fori_loop(0, nrows * cpr, body, 0)


def _zero_vmem_1d(ref, n):
    z = jnp.zeros((16,), jnp.float32)

    def body(i, _):
        ref[pl.ds(i * 16, 16)] = z
        return 0

    lax.fori_loop(0, n // 16, body, 0)


@functools.partial(
    pl.kernel,
    out_type=jax.ShapeDtypeStruct((NSC, NP), jnp.float32),
    mesh=_MESH,
    scratch_types=[
        pltpu.VMEM((NCHUNK, CH), jnp.int32),
        pltpu.VMEM((CH,), jnp.float32),
        pltpu.VMEM((RPT,), jnp.float32),
        pltpu.VMEM_SHARED((NP,), jnp.float32),
        pltpu.SemaphoreType.DMA,
    ],
)
def _deg_kernel(adj_hbm, out_hbm, cidx_v, ones_v, zbuf_v, deg_sh, sem):
    cid = lax.axis_index("c")
    sid = lax.axis_index("s")
    tile = cid * NTILE + sid

    _zero_vmem_1d(zbuf_v, RPT)
    o = jnp.ones((16,), jnp.float32)
    for i in range(CH // 16):
        ones_v[pl.ds(i * 16, 16)] = o
    pltpu.sync_copy(adj_hbm.at[1, tile], cidx_v)
    pltpu.sync_copy(zbuf_v, deg_sh.at[pl.ds(sid * RPT, RPT)])
    plsc.subcore_barrier()

    def fire(i, _):
        pltpu.sync_copy(ones_v, deg_sh.at[cidx_v.at[i]], add=True)
        return 0

    lax.fori_loop(0, NCHUNK, fire, 0)
    plsc.subcore_barrier()
    pltpu.sync_copy(deg_sh.at[pl.ds(sid * RPT, RPT)],
                    out_hbm.at[cid, pl.ds(sid * RPT, RPT)])


def _prep_body(deg_ref, x_ref, ew_ref, xs_ref, ew2_ref):
    d = deg_ref[0] + deg_ref[1]
    dis = jnp.where(d > 0.0, lax.rsqrt(jnp.maximum(d, 1e-30)), 0.0)
    xs_ref[...] = x_ref[...] * dis
    ew2_ref[...] = ew_ref[...] * dis


def _prep(deg2, x, ew):
    blk = 2000
    grid = N // blk
    return pl.pallas_call(
        _prep_body,
        grid=(grid,),
        in_specs=[
            pl.BlockSpec((NSC, blk, 1), lambda i: (0, i, 0)),
            pl.BlockSpec((blk, F), lambda i: (i, 0)),
            pl.BlockSpec((blk, NENV), lambda i: (i, 0)),
        ],
        out_specs=[
            pl.BlockSpec((blk, F), lambda i: (i, 0)),
            pl.BlockSpec((blk, NENV), lambda i: (i, 0)),
        ],
        out_shape=[
            jax.ShapeDtypeStruct((N, F), jnp.float32),
            jax.ShapeDtypeStruct((N, NENV), jnp.float32),
        ],
    )(deg2, x, ew)


@functools.partial(
    pl.kernel,
    out_type=jax.ShapeDtypeStruct((NSC, NP, F), jnp.float32),
    mesh=_MESH,
    scratch_types=[
        pltpu.VMEM((NCHUNK, CH), jnp.int32),
        pltpu.VMEM((CH,), jnp.int32),
        pltpu.VMEM((CH,), jnp.int32),
        pltpu.VMEM((CH, F), jnp.float32),
        pltpu.VMEM((CH, F), jnp.float32),
        pltpu.VMEM_SHARED((NP, F), jnp.float32),
        pltpu.SemaphoreType.DMA,
        pltpu.SemaphoreType.DMA,
        pltpu.SemaphoreType.DMA,
        pltpu.SemaphoreType.DMA,
        pltpu.SemaphoreType.DMA,
        pltpu.SemaphoreType.DMA,
    ],
)
def _scatter_kernel(adj_hbm, xs_hbm, out_hbm,
                    ridx_v, cidx0_v, cidx1_v, rows0_v, rows1_v, acc_sh,
                    gsem0, gsem1, csem0, csem1, ssem0, ssem1):
    cid = lax.axis_index("c")
    sid = lax.axis_index("s")
    tile = cid * NTILE + sid

    _zero_vmem_2d(rows0_v, CH, F)
    pltpu.sync_copy(adj_hbm.at[0, tile], ridx_v)
    for j in range(RPT // CH):
        pltpu.sync_copy(rows0_v, acc_sh.at[pl.ds(sid * RPT + j * CH, CH)])
    plsc.subcore_barrier()

    rbufs = (rows0_v, rows1_v)
    gsems = (gsem0, gsem1)
    cbufs = (cidx0_v, cidx1_v)
    csems = (csem0, csem1)
    ssems = (ssem0, ssem1)

    def start_gather(i, b):
        pltpu.async_copy(xs_hbm.at[ridx_v.at[i]], rbufs[b], gsems[b])

    def wait_gather(i, b):
        pltpu.make_async_copy(xs_hbm.at[ridx_v.at[i]], rbufs[b], gsems[b]).wait()

    def start_cidx(i, b):
        pltpu.async_copy(adj_hbm.at[1, tile, i], cbufs[b], csems[b])

    def wait_cidx(i, b):
        pltpu.make_async_copy(adj_hbm.at[1, tile, i], cbufs[b], csems[b]).wait()

    def fire_scatter(b):
        pltpu.async_copy(rbufs[b], acc_sh.at[cbufs[b]], ssems[b], add=True)

    def drain_scatter(b):
        pltpu.make_async_copy(rbufs[b], acc_sh.at[cbufs[b]], ssems[b]).wait()

    start_cidx(0, 0)
    start_gather(0, 0)

    def body(g, _):
        for b in range(2):
            i = g * 2 + b
            q = 1 - b

            @pl.when(i < NCHUNK)
            def _():
                wait_gather(i, b)
                wait_cidx(i, b)
                fire_scatter(b)

                @pl.when(i + 1 < NCHUNK)
                def _():
                    @pl.when(i >= 1)
                    def _():
                        drain_scatter(q)

                    start_cidx(i + 1, q)
                    start_gather(i + 1, q)

        return 0

    lax.fori_loop(0, (NCHUNK + 1) // 2, body, 0)
    drain_scatter((NCHUNK - 2) % 2)
    drain_scatter((NCHUNK - 1) % 2)
    plsc.subcore_barrier()
    pltpu.sync_copy(acc_sh.at[pl.ds(sid * RPT, RPT)],
                    out_hbm.at[cid, pl.ds(sid * RPT, RPT)])


def _final_body(acc_ref, x_ref, ew_ref, ew2_ref, w1_ref, w2_ref, out_ref):
    a = acc_ref[0] + acc_ref[1]
    xv = x_ref[...]
    A = jnp.dot(a.astype(jnp.bfloat16), w1_ref[...].astype(jnp.bfloat16),
                preferred_element_type=jnp.float32)
    Bm = jnp.dot(xv.astype(jnp.bfloat16), w2_ref[...].astype(jnp.bfloat16),
                 preferred_element_type=jnp.float32)
    o = xv
    for e in range(NENV):
        o = o + ew2_ref[:, e:e + 1] * A[:, F * e:F * (e + 1)]
        o = o + ew_ref[:, e:e + 1] * Bm[:, F * e:F * (e + 1)]
    out_ref[...] = o


def _final(acc2, x, ew, ew2, w1, w2):
    blk = 2000
    grid = N // blk
    return pl.pallas_call(
        _final_body,
        grid=(grid,),
        in_specs=[
            pl.BlockSpec((NSC, blk, F), lambda i: (0, i, 0)),
            pl.BlockSpec((blk, F), lambda i: (i, 0)),
            pl.BlockSpec((blk, NENV), lambda i: (i, 0)),
            pl.BlockSpec((blk, NENV), lambda i: (i, 0)),
            pl.BlockSpec((F, NENV * F), lambda i: (0, 0)),
            pl.BlockSpec((F, NENV * F), lambda i: (0, 0)),
        ],
        out_specs=pl.BlockSpec((blk, F), lambda i: (i, 0)),
        out_shape=jax.ShapeDtypeStruct((N, F), jnp.float32),
    )(acc2, x, ew, ew2, w1, w2)


def kernel(x, adj, env_weights, weights):
    adjr = adj.astype(jnp.int32).reshape(2, NSC * NTILE, TPE)
    pad = jnp.stack([
        jnp.zeros((NSC * NTILE, PADE), jnp.int32),
        jnp.full((NSC * NTILE, PADE), NP - 1, jnp.int32),
    ])
    adjp = jnp.concatenate([adjr, pad], axis=2).reshape(
        2, NSC * NTILE, NCHUNK, CH)

    deg2 = _deg_kernel(adjp)
    xs, ew2 = _prep(deg2.reshape(NSC, NP, 1), x, env_weights)
    acc2 = _scatter_kernel(adjp, xs)

    w1 = jnp.transpose(weights[:, :F, :], (1, 0, 2)).reshape(F, NENV * F)
    w2 = jnp.transpose(weights[:, F:, :], (1, 0, 2)).reshape(F, NENV * F)
    return _final(acc2, x, env_weights, ew2, w1, w2)


# speedup vs baseline: 1.0266x; 1.0266x over previous
import functools

import jax
import jax.numpy as jnp
from jax import lax
from jax.experimental import pallas as pl
from jax.experimental.pallas import tpu as pltpu
from jax.experimental.pallas import tpu_sc as plsc

N = 10000
E = 320000
F = 128
NENV = 4
NP = 10240
NSC = 2
NTILE = 16
TPE = E // (NSC * NTILE)
CH = 40
NCHUNK = TPE // CH
RPT = NP // NTILE

_MESH = plsc.VectorSubcoreMesh(core_axis_name="c", subcore_axis_name="s")


def _zero_vmem_2d(ref, nrows, ncols):
    z = jnp.zeros((16,), jnp.float32)
    cpr = ncols // 16

    def body(i, _):
        r = i // cpr
        c = i % cpr
        ref[r, pl.ds(c * 16, 16)] = z
        return 0

    lax.fori_loop(0, nrows * cpr, body, 0)


def _zero_vmem_1d(ref, n):
    z = jnp.zeros((16,), jnp.float32)

    def body(i, _):
        ref[pl.ds(i * 16, 16)] = z
        return 0

    lax.fori_loop(0, n // 16, body, 0)


@functools.partial(
    pl.kernel,
    out_type=jax.ShapeDtypeStruct((NSC, NP), jnp.float32),
    mesh=_MESH,
    scratch_types=[
        pltpu.VMEM((NCHUNK, CH), jnp.int32),
        pltpu.VMEM((CH,), jnp.float32),
        pltpu.VMEM((RPT,), jnp.float32),
        pltpu.VMEM_SHARED((NP,), jnp.float32),
        pltpu.SemaphoreType.DMA,
    ],
)
def _deg_kernel(col_hbm, out_hbm, cidx_v, ones_v, zbuf_v, deg_sh, sem):
    cid = lax.axis_index("c")
    sid = lax.axis_index("s")
    tile = cid * NTILE + sid

    _zero_vmem_1d(zbuf_v, RPT)
    o = jnp.ones((16,), jnp.float32)
    for i in range(CH // 16):
        ones_v[pl.ds(i * 16, 16)] = o
    pltpu.sync_copy(col_hbm.at[tile], cidx_v)
    pltpu.sync_copy(zbuf_v, deg_sh.at[pl.ds(sid * RPT, RPT)])
    plsc.subcore_barrier()

    def fire(i, _):
        pltpu.sync_copy(ones_v, deg_sh.at[cidx_v.at[i]], add=True)
        return 0

    lax.fori_loop(0, NCHUNK, fire, 0)
    plsc.subcore_barrier()
    pltpu.sync_copy(deg_sh.at[pl.ds(sid * RPT, RPT)],
                    out_hbm.at[cid, pl.ds(sid * RPT, RPT)])


def _prep_body(deg_ref, x_ref, ew_ref, xs_ref, ew2_ref):
    d = deg_ref[0] + deg_ref[1]
    dis = jnp.where(d > 0.0, lax.rsqrt(jnp.maximum(d, 1e-30)), 0.0)
    xs_ref[...] = x_ref[...] * dis
    ew2_ref[...] = ew_ref[...] * dis


def _prep(deg2, x, ew):
    blk = 2000
    grid = N // blk
    return pl.pallas_call(
        _prep_body,
        grid=(grid,),
        in_specs=[
            pl.BlockSpec((NSC, blk, 1), lambda i: (0, i, 0)),
            pl.BlockSpec((blk, F), lambda i: (i, 0)),
            pl.BlockSpec((blk, NENV), lambda i: (i, 0)),
        ],
        out_specs=[
            pl.BlockSpec((blk, F), lambda i: (i, 0)),
            pl.BlockSpec((blk, NENV), lambda i: (i, 0)),
        ],
        out_shape=[
            jax.ShapeDtypeStruct((N, F), jnp.float32),
            jax.ShapeDtypeStruct((N, NENV), jnp.float32),
        ],
    )(deg2, x, ew)


@functools.partial(
    pl.kernel,
    out_type=jax.ShapeDtypeStruct((NSC, NP, F), jnp.float32),
    mesh=_MESH,
    scratch_types=[
        pltpu.VMEM((NCHUNK, CH), jnp.int32),
        pltpu.VMEM((CH,), jnp.int32),
        pltpu.VMEM((CH,), jnp.int32),
        pltpu.VMEM((CH, F), jnp.float32),
        pltpu.VMEM((CH, F), jnp.float32),
        pltpu.VMEM_SHARED((NP, F), jnp.float32),
        pltpu.SemaphoreType.DMA,
        pltpu.SemaphoreType.DMA,
        pltpu.SemaphoreType.DMA,
        pltpu.SemaphoreType.DMA,
        pltpu.SemaphoreType.DMA,
        pltpu.SemaphoreType.DMA,
    ],
)
def _scatter_kernel(row_hbm, col_hbm, xs_hbm, out_hbm,
                    ridx_v, cidx0_v, cidx1_v, rows0_v, rows1_v, acc_sh,
                    gsem0, gsem1, csem0, csem1, ssem0, ssem1):
    cid = lax.axis_index("c")
    sid = lax.axis_index("s")
    tile = cid * NTILE + sid

    _zero_vmem_2d(rows0_v, CH, F)
    pltpu.sync_copy(row_hbm.at[tile], ridx_v)
    for j in range(RPT // CH):
        pltpu.sync_copy(rows0_v, acc_sh.at[pl.ds(sid * RPT + j * CH, CH)])
    plsc.subcore_barrier()

    rbufs = (rows0_v, rows1_v)
    gsems = (gsem0, gsem1)
    cbufs = (cidx0_v, cidx1_v)
    csems = (csem0, csem1)
    ssems = (ssem0, ssem1)

    def start_gather(i, b):
        pltpu.async_copy(xs_hbm.at[ridx_v.at[i]], rbufs[b], gsems[b])

    def wait_gather(i, b):
        pltpu.make_async_copy(xs_hbm.at[ridx_v.at[i]], rbufs[b], gsems[b]).wait()

    def start_cidx(i, b):
        pltpu.async_copy(col_hbm.at[tile, i], cbufs[b], csems[b])

    def wait_cidx(i, b):
        pltpu.make_async_copy(col_hbm.at[tile, i], cbufs[b], csems[b]).wait()

    def fire_scatter(b):
        pltpu.async_copy(rbufs[b], acc_sh.at[cbufs[b]], ssems[b], add=True)

    def drain_scatter(b):
        pltpu.make_async_copy(rbufs[b], acc_sh.at[cbufs[b]], ssems[b]).wait()

    start_cidx(0, 0)
    start_gather(0, 0)

    def body(g, _):
        for b in range(2):
            i = g * 2 + b
            q = 1 - b

            @pl.when(i < NCHUNK)
            def _():
                wait_gather(i, b)
                wait_cidx(i, b)
                fire_scatter(b)

                @pl.when(i + 1 < NCHUNK)
                def _():
                    @pl.when(i >= 1)
                    def _():
                        drain_scatter(q)

                    start_cidx(i + 1, q)
                    start_gather(i + 1, q)

        return 0

    lax.fori_loop(0, (NCHUNK + 1) // 2, body, 0)
    drain_scatter((NCHUNK - 2) % 2)
    drain_scatter((NCHUNK - 1) % 2)
    plsc.subcore_barrier()
    pltpu.sync_copy(acc_sh.at[pl.ds(sid * RPT, RPT)],
                    out_hbm.at[cid, pl.ds(sid * RPT, RPT)])


def _final_body(acc_ref, x_ref, ew_ref, ew2_ref, w1_ref, w2_ref, out_ref):
    a = acc_ref[0] + acc_ref[1]
    xv = x_ref[...]
    A = jnp.dot(a, w1_ref[...], preferred_element_type=jnp.float32)
    Bm = jnp.dot(xv, w2_ref[...], preferred_element_type=jnp.float32)
    o = xv
    for e in range(NENV):
        o = o + ew2_ref[:, e:e + 1] * A[:, F * e:F * (e + 1)]
        o = o + ew_ref[:, e:e + 1] * Bm[:, F * e:F * (e + 1)]
    out_ref[...] = o


def _final(acc2, x, ew, ew2, w1, w2):
    blk = 2000
    grid = N // blk
    return pl.pallas_call(
        _final_body,
        grid=(grid,),
        in_specs=[
            pl.BlockSpec((NSC, blk, F), lambda i: (0, i, 0)),
            pl.BlockSpec((blk, F), lambda i: (i, 0)),
            pl.BlockSpec((blk, NENV), lambda i: (i, 0)),
            pl.BlockSpec((blk, NENV), lambda i: (i, 0)),
            pl.BlockSpec((F, NENV * F), lambda i: (0, 0)),
            pl.BlockSpec((F, NENV * F), lambda i: (0, 0)),
        ],
        out_specs=pl.BlockSpec((blk, F), lambda i: (i, 0)),
        out_shape=jax.ShapeDtypeStruct((N, F), jnp.float32),
    )(acc2, x, ew, ew2, w1, w2)


def kernel(x, adj, env_weights, weights):
    row = adj[0].astype(jnp.int32).reshape(NSC * NTILE, NCHUNK, CH)
    col = adj[1].astype(jnp.int32).reshape(NSC * NTILE, NCHUNK, CH)

    deg2 = _deg_kernel(col)
    xs, ew2 = _prep(deg2.reshape(NSC, NP, 1), x, env_weights)
    acc2 = _scatter_kernel(row, col, xs)

    w1 = jnp.transpose(weights[:, :F, :], (1, 0, 2)).reshape(F, NENV * F)
    w2 = jnp.transpose(weights[:, F:, :], (1, 0, 2)).reshape(F, NENV * F)
    return _final(acc2, x, env_weights, ew2, w1, w2)


# speedup vs baseline: 1.3709x; 1.3353x over previous
import functools

import jax
import jax.numpy as jnp
from jax import lax
from jax.experimental import pallas as pl
from jax.experimental.pallas import tpu as pltpu
from jax.experimental.pallas import tpu_sc as plsc

N = 10000
E = 320000
F = 128
NENV = 4
NP = 10240
NSC = 2
NTILE = 16
TPE = E // (NSC * NTILE)
CH = 80
NCHUNK = TPE // CH
RPT = NP // NTILE

_MESH = plsc.VectorSubcoreMesh(core_axis_name="c", subcore_axis_name="s")


def _zero_vmem_2d(ref, nrows, ncols):
    z = jnp.zeros((16,), jnp.float32)
    cpr = ncols // 16

    def body(i, _):
        r = i // cpr
        c = i % cpr
        ref[r, pl.ds(c * 16, 16)] = z
        return 0

    lax.fori_loop(0, nrows * cpr, body, 0)


def _zero_vmem_1d(ref, n):
    z = jnp.zeros((16,), jnp.float32)

    def body(i, _):
        ref[pl.ds(i * 16, 16)] = z
        return 0

    lax.fori_loop(0, n // 16, body, 0)


@functools.partial(
    pl.kernel,
    out_type=jax.ShapeDtypeStruct((NSC, NP), jnp.float32),
    mesh=_MESH,
    scratch_types=[
        pltpu.VMEM((NCHUNK, CH), jnp.int32),
        pltpu.VMEM((CH,), jnp.float32),
        pltpu.VMEM((RPT,), jnp.float32),
        pltpu.VMEM_SHARED((NP,), jnp.float32),
        pltpu.SemaphoreType.DMA,
    ],
)
def _deg_kernel(col_hbm, out_hbm, cidx_v, ones_v, zbuf_v, deg_sh, sem):
    cid = lax.axis_index("c")
    sid = lax.axis_index("s")
    tile = cid * NTILE + sid

    _zero_vmem_1d(zbuf_v, RPT)
    o = jnp.ones((16,), jnp.float32)
    for i in range(CH // 16):
        ones_v[pl.ds(i * 16, 16)] = o
    pltpu.sync_copy(col_hbm.at[tile], cidx_v)
    pltpu.sync_copy(zbuf_v, deg_sh.at[pl.ds(sid * RPT, RPT)])
    plsc.subcore_barrier()

    def fire(i, _):
        pltpu.sync_copy(ones_v, deg_sh.at[cidx_v.at[i]], add=True)
        return 0

    lax.fori_loop(0, NCHUNK, fire, 0)
    plsc.subcore_barrier()
    pltpu.sync_copy(deg_sh.at[pl.ds(sid * RPT, RPT)],
                    out_hbm.at[cid, pl.ds(sid * RPT, RPT)])


def _prep_body(deg_ref, x_ref, xs_ref):
    d = deg_ref[0] + deg_ref[1]
    dis = jnp.where(d > 0.0, lax.rsqrt(jnp.maximum(d, 1e-30)), 0.0)
    xs_ref[...] = x_ref[...] * dis


def _prep(deg2, x):
    blk = 1000
    grid = N // blk
    return pl.pallas_call(
        _prep_body,
        grid=(grid,),
        in_specs=[
            pl.BlockSpec((NSC, blk, 1), lambda i: (0, i, 0)),
            pl.BlockSpec((blk, F), lambda i: (i, 0)),
        ],
        out_specs=pl.BlockSpec((blk, F), lambda i: (i, 0)),
        out_shape=jax.ShapeDtypeStruct((N, F), jnp.float32),
    )(deg2, x)


@functools.partial(
    pl.kernel,
    out_type=jax.ShapeDtypeStruct((NSC, NP, F), jnp.float32),
    mesh=_MESH,
    scratch_types=[
        pltpu.VMEM((NCHUNK, CH), jnp.int32),
        pltpu.VMEM((CH,), jnp.int32),
        pltpu.VMEM((CH,), jnp.int32),
        pltpu.VMEM((CH, F), jnp.float32),
        pltpu.VMEM((CH, F), jnp.float32),
        pltpu.VMEM_SHARED((NP, F), jnp.float32),
        pltpu.SemaphoreType.DMA,
        pltpu.SemaphoreType.DMA,
        pltpu.SemaphoreType.DMA,
        pltpu.SemaphoreType.DMA,
        pltpu.SemaphoreType.DMA,
        pltpu.SemaphoreType.DMA,
    ],
)
def _scatter_kernel(row_hbm, col_hbm, xs_hbm, out_hbm,
                    ridx_v, cidx0_v, cidx1_v, rows0_v, rows1_v, acc_sh,
                    gsem0, gsem1, csem0, csem1, ssem0, ssem1):
    cid = lax.axis_index("c")
    sid = lax.axis_index("s")
    tile = cid * NTILE + sid

    _zero_vmem_2d(rows0_v, CH, F)
    pltpu.sync_copy(row_hbm.at[tile], ridx_v)
    for j in range(RPT // CH):
        pltpu.sync_copy(rows0_v, acc_sh.at[pl.ds(sid * RPT + j * CH, CH)])
    plsc.subcore_barrier()

    rbufs = (rows0_v, rows1_v)
    gsems = (gsem0, gsem1)
    cbufs = (cidx0_v, cidx1_v)
    csems = (csem0, csem1)
    ssems = (ssem0, ssem1)

    def start_gather(i, b):
        pltpu.async_copy(xs_hbm.at[ridx_v.at[i]], rbufs[b], gsems[b])

    def wait_gather(i, b):
        pltpu.make_async_copy(xs_hbm.at[ridx_v.at[i]], rbufs[b], gsems[b]).wait()

    def start_cidx(i, b):
        pltpu.async_copy(col_hbm.at[tile, i], cbufs[b], csems[b])

    def wait_cidx(i, b):
        pltpu.make_async_copy(col_hbm.at[tile, i], cbufs[b], csems[b]).wait()

    def fire_scatter(b):
        pltpu.async_copy(rbufs[b], acc_sh.at[cbufs[b]], ssems[b], add=True)

    def drain_scatter(b):
        pltpu.make_async_copy(rbufs[b], acc_sh.at[cbufs[b]], ssems[b]).wait()

    start_cidx(0, 0)
    start_gather(0, 0)

    def body(g, _):
        for b in range(2):
            i = g * 2 + b
            q = 1 - b

            @pl.when(i < NCHUNK)
            def _():
                wait_gather(i, b)
                wait_cidx(i, b)
                fire_scatter(b)

                @pl.when(i + 1 < NCHUNK)
                def _():
                    @pl.when(i >= 1)
                    def _():
                        drain_scatter(q)

                    start_cidx(i + 1, q)
                    start_gather(i + 1, q)

        return 0

    lax.fori_loop(0, (NCHUNK + 1) // 2, body, 0)
    drain_scatter((NCHUNK - 2) % 2)
    drain_scatter((NCHUNK - 1) % 2)
    plsc.subcore_barrier()
    pltpu.sync_copy(acc_sh.at[pl.ds(sid * RPT, RPT)],
                    out_hbm.at[cid, pl.ds(sid * RPT, RPT)])


def _final_body(acc_ref, deg_ref, x_ref, ew_ref, w_ref, out_ref):
    a = acc_ref[0] + acc_ref[1]
    d = deg_ref[0] + deg_ref[1]
    dis = jnp.where(d > 0.0, lax.rsqrt(jnp.maximum(d, 1e-30)), 0.0)
    xv = x_ref[...]
    o = xv
    for e in range(NENV):
        A = jnp.dot(a, w_ref[e, :F, :], preferred_element_type=jnp.float32)
        Bm = jnp.dot(xv, w_ref[e, F:, :], preferred_element_type=jnp.float32)
        we = ew_ref[:, e:e + 1]
        o = o + (we * dis) * A + we * Bm
    out_ref[...] = o


def _final(acc2, deg2, x, ew, w):
    blk = 1000
    grid = N // blk
    return pl.pallas_call(
        _final_body,
        grid=(grid,),
        in_specs=[
            pl.BlockSpec((NSC, blk, F), lambda i: (0, i, 0)),
            pl.BlockSpec((NSC, blk, 1), lambda i: (0, i, 0)),
            pl.BlockSpec((blk, F), lambda i: (i, 0)),
            pl.BlockSpec((blk, NENV), lambda i: (i, 0)),
            pl.BlockSpec((NENV, 2 * F, F), lambda i: (0, 0, 0)),
        ],
        out_specs=pl.BlockSpec((blk, F), lambda i: (i, 0)),
        out_shape=jax.ShapeDtypeStruct((N, F), jnp.float32),
    )(acc2, deg2, x, ew, w)


def kernel(x, adj, env_weights, weights):
    row = adj[0].astype(jnp.int32).reshape(NSC * NTILE, NCHUNK, CH)
    col = adj[1].astype(jnp.int32).reshape(NSC * NTILE, NCHUNK, CH)

    deg2 = _deg_kernel(col)
    deg3 = deg2.reshape(NSC, NP, 1)
    xs = _prep(deg3, x)
    acc2 = _scatter_kernel(row, col, xs)
    return _final(acc2, deg3, x, env_weights, weights)


# speedup vs baseline: 1.3926x; 1.0158x over previous
import functools

import jax
import jax.numpy as jnp
from jax import lax
from jax.experimental import pallas as pl
from jax.experimental.pallas import tpu as pltpu
from jax.experimental.pallas import tpu_sc as plsc

N = 10000
E = 320000
F = 128
NENV = 4
NP = 10240
NSC = 2
NTILE = 16
TPE = E // (NSC * NTILE)
CH = 80
NCHUNK = TPE // CH
RPT = NP // NTILE

_MESH = plsc.VectorSubcoreMesh(core_axis_name="c", subcore_axis_name="s")


def _zero_vmem_2d(ref, nrows, ncols):
    z = jnp.zeros((16,), jnp.float32)
    cpr = ncols // 16

    def body(i, _):
        r = i // cpr
        c = i % cpr
        ref[r, pl.ds(c * 16, 16)] = z
        return 0

    lax.fori_loop(0, nrows * cpr, body, 0)


def _zero_vmem_1d(ref, n):
    z = jnp.zeros((16,), jnp.float32)

    def body(i, _):
        ref[pl.ds(i * 16, 16)] = z
        return 0

    lax.fori_loop(0, n // 16, body, 0)


@functools.partial(
    pl.kernel,
    out_type=jax.ShapeDtypeStruct((NSC, NP), jnp.float32),
    mesh=_MESH,
    scratch_types=[
        pltpu.VMEM((NCHUNK, CH), jnp.int32),
        pltpu.VMEM((CH,), jnp.float32),
        pltpu.VMEM((RPT,), jnp.float32),
        pltpu.VMEM_SHARED((NP,), jnp.float32),
        pltpu.SemaphoreType.DMA,
    ],
)
def _deg_kernel(col_hbm, out_hbm, cidx_v, ones_v, zbuf_v, deg_sh, sem):
    cid = lax.axis_index("c")
    sid = lax.axis_index("s")
    tile = cid * NTILE + sid

    _zero_vmem_1d(zbuf_v, RPT)
    o = jnp.ones((16,), jnp.float32)
    for i in range(CH // 16):
        ones_v[pl.ds(i * 16, 16)] = o
    pltpu.sync_copy(col_hbm.at[tile], cidx_v)
    pltpu.sync_copy(zbuf_v, deg_sh.at[pl.ds(sid * RPT, RPT)])
    plsc.subcore_barrier()

    def fire(i, _):
        pltpu.sync_copy(ones_v, deg_sh.at[cidx_v.at[i]], add=True)
        return 0

    lax.fori_loop(0, NCHUNK, fire, 0)
    plsc.subcore_barrier()
    pltpu.sync_copy(deg_sh.at[pl.ds(sid * RPT, RPT)],
                    out_hbm.at[cid, pl.ds(sid * RPT, RPT)])


def _prep_body(deg_ref, x_ref, xs_ref):
    d = deg_ref[0] + deg_ref[1]
    dis = jnp.where(d > 0.0, lax.rsqrt(jnp.maximum(d, 1e-30)), 0.0)
    xs_ref[...] = x_ref[...] * dis


def _prep(deg2, x):
    blk = 2000
    grid = N // blk
    return pl.pallas_call(
        _prep_body,
        grid=(grid,),
        in_specs=[
            pl.BlockSpec((NSC, blk, 1), lambda i: (0, i, 0)),
            pl.BlockSpec((blk, F), lambda i: (i, 0)),
        ],
        out_specs=pl.BlockSpec((blk, F), lambda i: (i, 0)),
        out_shape=jax.ShapeDtypeStruct((N, F), jnp.float32),
    )(deg2, x)


@functools.partial(
    pl.kernel,
    out_type=jax.ShapeDtypeStruct((NSC, NP, F), jnp.float32),
    mesh=_MESH,
    scratch_types=[
        pltpu.VMEM((NCHUNK, CH), jnp.int32),
        pltpu.VMEM((CH,), jnp.int32),
        pltpu.VMEM((CH,), jnp.int32),
        pltpu.VMEM((CH, F), jnp.float32),
        pltpu.VMEM((CH, F), jnp.float32),
        pltpu.VMEM_SHARED((NP, F), jnp.float32),
        pltpu.SemaphoreType.DMA,
        pltpu.SemaphoreType.DMA,
        pltpu.SemaphoreType.DMA,
        pltpu.SemaphoreType.DMA,
        pltpu.SemaphoreType.DMA,
        pltpu.SemaphoreType.DMA,
    ],
)
def _scatter_kernel(row_hbm, col_hbm, xs_hbm, out_hbm,
                    ridx_v, cidx0_v, cidx1_v, rows0_v, rows1_v, acc_sh,
                    gsem0, gsem1, csem0, csem1, ssem0, ssem1):
    cid = lax.axis_index("c")
    sid = lax.axis_index("s")
    tile = cid * NTILE + sid

    _zero_vmem_2d(rows0_v, CH, F)
    pltpu.sync_copy(row_hbm.at[tile], ridx_v)
    for j in range(RPT // CH):
        pltpu.sync_copy(rows0_v, acc_sh.at[pl.ds(sid * RPT + j * CH, CH)])
    plsc.subcore_barrier()

    rbufs = (rows0_v, rows1_v)
    gsems = (gsem0, gsem1)
    cbufs = (cidx0_v, cidx1_v)
    csems = (csem0, csem1)
    ssems = (ssem0, ssem1)

    def start_gather(i, b):
        pltpu.async_copy(xs_hbm.at[ridx_v.at[i]], rbufs[b], gsems[b])

    def wait_gather(i, b):
        pltpu.make_async_copy(xs_hbm.at[ridx_v.at[i]], rbufs[b], gsems[b]).wait()

    def start_cidx(i, b):
        pltpu.async_copy(col_hbm.at[tile, i], cbufs[b], csems[b])

    def wait_cidx(i, b):
        pltpu.make_async_copy(col_hbm.at[tile, i], cbufs[b], csems[b]).wait()

    def fire_scatter(b):
        pltpu.async_copy(rbufs[b], acc_sh.at[cbufs[b]], ssems[b], add=True)

    def drain_scatter(b):
        pltpu.make_async_copy(rbufs[b], acc_sh.at[cbufs[b]], ssems[b]).wait()

    start_cidx(0, 0)
    start_gather(0, 0)

    def body(g, _):
        for b in range(2):
            i = g * 2 + b
            q = 1 - b

            @pl.when(i < NCHUNK)
            def _():
                wait_gather(i, b)
                wait_cidx(i, b)
                fire_scatter(b)

                @pl.when(i + 1 < NCHUNK)
                def _():
                    @pl.when(i >= 1)
                    def _():
                        drain_scatter(q)

                    start_cidx(i + 1, q)
                    start_gather(i + 1, q)

        return 0

    lax.fori_loop(0, (NCHUNK + 1) // 2, body, 0)
    drain_scatter((NCHUNK - 2) % 2)
    drain_scatter((NCHUNK - 1) % 2)
    plsc.subcore_barrier()
    pltpu.sync_copy(acc_sh.at[pl.ds(sid * RPT, RPT)],
                    out_hbm.at[cid, pl.ds(sid * RPT, RPT)])


def _final_body(acc_ref, deg_ref, x_ref, ew_ref, w1_ref, w2_ref, out_ref):
    a = acc_ref[0] + acc_ref[1]
    d = deg_ref[0] + deg_ref[1]
    dis = jnp.where(d > 0.0, lax.rsqrt(jnp.maximum(d, 1e-30)), 0.0)
    xv = x_ref[...]
    A = jnp.dot(a, w1_ref[...], preferred_element_type=jnp.float32)
    Bm = jnp.dot(xv, w2_ref[...], preferred_element_type=jnp.float32)
    o = xv
    for e in range(NENV):
        we = ew_ref[:, e:e + 1]
        o = o + (we * dis) * A[:, F * e:F * (e + 1)]
        o = o + we * Bm[:, F * e:F * (e + 1)]
    out_ref[...] = o


def _final(acc2, deg2, x, ew, w1, w2):
    blk = 2000
    grid = N // blk
    return pl.pallas_call(
        _final_body,
        grid=(grid,),
        in_specs=[
            pl.BlockSpec((NSC, blk, F), lambda i: (0, i, 0)),
            pl.BlockSpec((NSC, blk, 1), lambda i: (0, i, 0)),
            pl.BlockSpec((blk, F), lambda i: (i, 0)),
            pl.BlockSpec((blk, NENV), lambda i: (i, 0)),
            pl.BlockSpec((F, NENV * F), lambda i: (0, 0)),
            pl.BlockSpec((F, NENV * F), lambda i: (0, 0)),
        ],
        out_specs=pl.BlockSpec((blk, F), lambda i: (i, 0)),
        out_shape=jax.ShapeDtypeStruct((N, F), jnp.float32),
    )(acc2, deg2, x, ew, w1, w2)


def kernel(x, adj, env_weights, weights):
    row = adj[0].astype(jnp.int32).reshape(NSC * NTILE, NCHUNK, CH)
    col = adj[1].astype(jnp.int32).reshape(NSC * NTILE, NCHUNK, CH)

    deg2 = _deg_kernel(col)
    deg3 = deg2.reshape(NSC, NP, 1)
    xs = _prep(deg3, x)
    acc2 = _scatter_kernel(row, col, xs)

    w1 = jnp.transpose(weights[:, :F, :], (1, 0, 2)).reshape(F, NENV * F)
    w2 = jnp.transpose(weights[:, F:, :], (1, 0, 2)).reshape(F, NENV * F)
    return _final(acc2, deg3, x, env_weights, w1, w2)


# speedup vs baseline: 1.9158x; 1.3757x over previous
import functools

import jax
import jax.numpy as jnp
from jax import lax
from jax.experimental import pallas as pl
from jax.experimental.pallas import tpu as pltpu
from jax.experimental.pallas import tpu_sc as plsc

N = 10000
E = 320000
F = 128
NENV = 4
NP = 10240
NSC = 2
NTILE = 16
TPE = E // (NSC * NTILE)
CH = 80
NCHUNK = TPE // CH
RPT = NP // NTILE

_MESH = plsc.VectorSubcoreMesh(core_axis_name="c", subcore_axis_name="s")


def _zero_vmem_2d(ref, nrows, ncols):
    z = jnp.zeros((16,), jnp.float32)
    cpr = ncols // 16

    def body(i, _):
        r = i // cpr
        c = i % cpr
        ref[r, pl.ds(c * 16, 16)] = z
        return 0

    lax.fori_loop(0, nrows * cpr, body, 0)


def _zero_vmem_1d(ref, n):
    z = jnp.zeros((16,), jnp.float32)

    def body(i, _):
        ref[pl.ds(i * 16, 16)] = z
        return 0

    lax.fori_loop(0, n // 16, body, 0)


@functools.partial(
    pl.kernel,
    out_type=jax.ShapeDtypeStruct((NSC, NP), jnp.float32),
    mesh=_MESH,
    scratch_types=[
        pltpu.VMEM((NCHUNK, CH), jnp.int32),
        pltpu.VMEM((CH,), jnp.float32),
        pltpu.VMEM((RPT,), jnp.float32),
        pltpu.VMEM_SHARED((NP,), jnp.float32),
        pltpu.SemaphoreType.DMA,
    ],
)
def _deg_kernel(col_hbm, out_hbm, cidx_v, ones_v, zbuf_v, deg_sh, sem):
    cid = lax.axis_index("c")
    sid = lax.axis_index("s")
    tile = cid * NTILE + sid

    _zero_vmem_1d(zbuf_v, RPT)
    o = jnp.ones((16,), jnp.float32)
    for i in range(CH // 16):
        ones_v[pl.ds(i * 16, 16)] = o
    pltpu.sync_copy(col_hbm.at[tile], cidx_v)
    pltpu.sync_copy(zbuf_v, deg_sh.at[pl.ds(sid * RPT, RPT)])
    plsc.subcore_barrier()

    def fire(i, _):
        pltpu.async_copy(ones_v, deg_sh.at[cidx_v.at[i]], sem, add=True)
        return 0

    lax.fori_loop(0, NCHUNK, fire, 0)

    def drain(i, _):
        pltpu.make_async_copy(ones_v, deg_sh.at[cidx_v.at[0]], sem).wait()
        return 0

    lax.fori_loop(0, NCHUNK, drain, 0)
    plsc.subcore_barrier()
    pltpu.sync_copy(deg_sh.at[pl.ds(sid * RPT, RPT)],
                    out_hbm.at[cid, pl.ds(sid * RPT, RPT)])


def _prep_body(deg_ref, x_ref, xs_ref):
    d = deg_ref[0] + deg_ref[1]
    dis = jnp.where(d > 0.0, lax.rsqrt(jnp.maximum(d, 1e-30)), 0.0)
    xs_ref[...] = x_ref[...] * dis


def _prep(deg2, x):
    blk = 2000
    grid = N // blk
    return pl.pallas_call(
        _prep_body,
        grid=(grid,),
        in_specs=[
            pl.BlockSpec((NSC, blk, 1), lambda i: (0, i, 0)),
            pl.BlockSpec((blk, F), lambda i: (i, 0)),
        ],
        out_specs=pl.BlockSpec((blk, F), lambda i: (i, 0)),
        out_shape=jax.ShapeDtypeStruct((N, F), jnp.float32),
    )(deg2, x)


@functools.partial(
    pl.kernel,
    out_type=jax.ShapeDtypeStruct((NSC, NP, F), jnp.float32),
    mesh=_MESH,
    scratch_types=[
        pltpu.VMEM((NCHUNK, CH), jnp.int32),
        pltpu.VMEM((CH,), jnp.int32),
        pltpu.VMEM((CH,), jnp.int32),
        pltpu.VMEM((CH,), jnp.int32),
        pltpu.VMEM((CH, F), jnp.float32),
        pltpu.VMEM((CH, F), jnp.float32),
        pltpu.VMEM((CH, F), jnp.float32),
        pltpu.VMEM_SHARED((NP, F), jnp.float32),
        pltpu.SemaphoreType.DMA,
        pltpu.SemaphoreType.DMA,
        pltpu.SemaphoreType.DMA,
        pltpu.SemaphoreType.DMA,
        pltpu.SemaphoreType.DMA,
        pltpu.SemaphoreType.DMA,
    ],
)
def _scatter_kernel(row_hbm, col_hbm, xs_hbm, out_hbm,
                    ridx_v, cidx0_v, cidx1_v, cidx2_v,
                    rows0_v, rows1_v, rows2_v, acc_sh,
                    gsem0, gsem1, gsem2, csem0, csem1, csem2):
    cid = lax.axis_index("c")
    sid = lax.axis_index("s")
    tile = cid * NTILE + sid

    _zero_vmem_2d(rows0_v, CH, F)
    pltpu.sync_copy(row_hbm.at[tile], ridx_v)
    for j in range(RPT // CH):
        pltpu.sync_copy(rows0_v, acc_sh.at[pl.ds(sid * RPT + j * CH, CH)])
    plsc.subcore_barrier()

    rbufs = (rows0_v, rows1_v, rows2_v)
    gsems = (gsem0, gsem1, gsem2)
    cbufs = (cidx0_v, cidx1_v, cidx2_v)
    csems = (csem0, csem1, csem2)

    def start_gather(i, b):
        pltpu.async_copy(xs_hbm.at[ridx_v.at[i]], rbufs[b], gsems[b])

    def wait_gather(i, b):
        pltpu.make_async_copy(xs_hbm.at[ridx_v.at[i]], rbufs[b], gsems[b]).wait()

    def start_cidx(i, b):
        pltpu.async_copy(col_hbm.at[tile, i], cbufs[b], csems[b])

    def wait_cidx(i, b):
        pltpu.make_async_copy(col_hbm.at[tile, i], cbufs[b], csems[b]).wait()

    start_cidx(0, 0)
    start_cidx(1, 1)
    start_gather(0, 0)
    start_gather(1, 1)

    def body(g, _):
        for b in range(3):
            i = g * 3 + b
            b2 = (b + 2) % 3

            @pl.when(i < NCHUNK)
            def _():
                wait_gather(i, b)

                @pl.when(i + 2 < NCHUNK)
                def _():
                    start_gather(i + 2, b2)
                    start_cidx(i + 2, b2)

                wait_cidx(i, b)
                pltpu.sync_copy(rbufs[b], acc_sh.at[cbufs[b]], add=True)
        return 0

    lax.fori_loop(0, (NCHUNK + 2) // 3, body, 0)
    plsc.subcore_barrier()
    pltpu.sync_copy(acc_sh.at[pl.ds(sid * RPT, RPT)],
                    out_hbm.at[cid, pl.ds(sid * RPT, RPT)])


def _final_body(acc_ref, deg_ref, x_ref, ew_ref, w1_ref, w2_ref, out_ref):
    a = acc_ref[0] + acc_ref[1]
    d = deg_ref[0] + deg_ref[1]
    dis = jnp.where(d > 0.0, lax.rsqrt(jnp.maximum(d, 1e-30)), 0.0)
    xv = x_ref[...]
    A = jnp.dot(a, w1_ref[...], preferred_element_type=jnp.float32)
    Bm = jnp.dot(xv, w2_ref[...], preferred_element_type=jnp.float32)
    o = xv
    for e in range(NENV):
        we = ew_ref[:, e:e + 1]
        o = o + (we * dis) * A[:, F * e:F * (e + 1)]
        o = o + we * Bm[:, F * e:F * (e + 1)]
    out_ref[...] = o


def _final(acc2, deg2, x, ew, w1, w2):
    blk = 2000
    grid = N // blk
    return pl.pallas_call(
        _final_body,
        grid=(grid,),
        in_specs=[
            pl.BlockSpec((NSC, blk, F), lambda i: (0, i, 0)),
            pl.BlockSpec((NSC, blk, 1), lambda i: (0, i, 0)),
            pl.BlockSpec((blk, F), lambda i: (i, 0)),
            pl.BlockSpec((blk, NENV), lambda i: (i, 0)),
            pl.BlockSpec((F, NENV * F), lambda i: (0, 0)),
            pl.BlockSpec((F, NENV * F), lambda i: (0, 0)),
        ],
        out_specs=pl.BlockSpec((blk, F), lambda i: (i, 0)),
        out_shape=jax.ShapeDtypeStruct((N, F), jnp.float32),
    )(acc2, deg2, x, ew, w1, w2)


def kernel(x, adj, env_weights, weights):
    row = adj[0].astype(jnp.int32).reshape(NSC * NTILE, NCHUNK, CH)
    col = adj[1].astype(jnp.int32).reshape(NSC * NTILE, NCHUNK, CH)

    deg2 = _deg_kernel(col)
    deg3 = deg2.reshape(NSC, NP, 1)
    xs = _prep(deg3, x)
    acc2 = _scatter_kernel(row, col, xs)

    w1 = jnp.transpose(weights[:, :F, :], (1, 0, 2)).reshape(F, NENV * F)
    w2 = jnp.transpose(weights[:, F:, :], (1, 0, 2)).reshape(F, NENV * F)
    return _final(acc2, deg3, x, env_weights, w1, w2)


# speedup vs baseline: 2.0215x; 1.0551x over previous
import functools

import jax
import jax.numpy as jnp
from jax import lax
from jax.experimental import pallas as pl
from jax.experimental.pallas import tpu as pltpu
from jax.experimental.pallas import tpu_sc as plsc

N = 10000
E = 320000
F = 128
NENV = 4
NP = 10240
NSC = 2
NTILE = 16
TPE = E // (NSC * NTILE)
CH = 80
NCHUNK = TPE // CH
RPT = NP // NTILE

_MESH = plsc.VectorSubcoreMesh(core_axis_name="c", subcore_axis_name="s")


def _zero_vmem_2d(ref, nrows, ncols):
    z = jnp.zeros((16,), jnp.float32)
    cpr = ncols // 16

    def body(i, _):
        r = i // cpr
        c = i % cpr
        ref[r, pl.ds(c * 16, 16)] = z
        return 0

    lax.fori_loop(0, nrows * cpr, body, 0)


def _zero_vmem_1d(ref, n):
    z = jnp.zeros((16,), jnp.float32)

    def body(i, _):
        ref[pl.ds(i * 16, 16)] = z
        return 0

    lax.fori_loop(0, n // 16, body, 0)


@functools.partial(
    pl.kernel,
    out_type=jax.ShapeDtypeStruct((NSC, NP), jnp.float32),
    mesh=_MESH,
    scratch_types=[
        pltpu.VMEM((NCHUNK, CH), jnp.int32),
        pltpu.VMEM((CH,), jnp.float32),
        pltpu.VMEM((RPT,), jnp.float32),
        pltpu.VMEM_SHARED((NP,), jnp.float32),
        pltpu.SemaphoreType.DMA,
    ],
)
def _deg_kernel(adj_hbm, out_hbm, cidx_v, ones_v, zbuf_v, deg_sh, sem):
    cid = lax.axis_index("c")
    sid = lax.axis_index("s")
    tile = cid * NTILE + sid

    _zero_vmem_1d(zbuf_v, RPT)
    o = jnp.ones((16,), jnp.float32)
    for i in range(CH // 16):
        ones_v[pl.ds(i * 16, 16)] = o
    pltpu.sync_copy(adj_hbm.at[1, tile], cidx_v)
    pltpu.sync_copy(zbuf_v, deg_sh.at[pl.ds(sid * RPT, RPT)])
    plsc.subcore_barrier()

    def fire(i, _):
        pltpu.async_copy(ones_v, deg_sh.at[cidx_v.at[i]], sem, add=True)
        return 0

    lax.fori_loop(0, NCHUNK, fire, 0)

    def drain(i, _):
        pltpu.make_async_copy(ones_v, deg_sh.at[cidx_v.at[0]], sem).wait()
        return 0

    lax.fori_loop(0, NCHUNK, drain, 0)
    plsc.subcore_barrier()
    pltpu.sync_copy(deg_sh.at[pl.ds(sid * RPT, RPT)],
                    out_hbm.at[cid, pl.ds(sid * RPT, RPT)])


def _prep_body(deg_ref, x_ref, xs_ref):
    d = deg_ref[0] + deg_ref[1]
    dis = jnp.where(d > 0.0, lax.rsqrt(jnp.maximum(d, 1e-30)), 0.0)
    xs_ref[...] = x_ref[...] * dis


def _prep(deg2, x):
    blk = 2000
    grid = N // blk
    return pl.pallas_call(
        _prep_body,
        grid=(grid,),
        in_specs=[
            pl.BlockSpec((NSC, blk, 1), lambda i: (0, i, 0)),
            pl.BlockSpec((blk, F), lambda i: (i, 0)),
        ],
        out_specs=pl.BlockSpec((blk, F), lambda i: (i, 0)),
        out_shape=jax.ShapeDtypeStruct((N, F), jnp.float32),
    )(deg2, x)


@functools.partial(
    pl.kernel,
    out_type=jax.ShapeDtypeStruct((NSC, NP, F), jnp.float32),
    mesh=_MESH,
    scratch_types=[
        pltpu.VMEM((NCHUNK, CH), jnp.int32),
        pltpu.VMEM((CH,), jnp.int32),
        pltpu.VMEM((CH,), jnp.int32),
        pltpu.VMEM((CH,), jnp.int32),
        pltpu.VMEM((CH, F), jnp.float32),
        pltpu.VMEM((CH, F), jnp.float32),
        pltpu.VMEM((CH, F), jnp.float32),
        pltpu.VMEM_SHARED((NP, F), jnp.float32),
        pltpu.SemaphoreType.DMA,
        pltpu.SemaphoreType.DMA,
        pltpu.SemaphoreType.DMA,
        pltpu.SemaphoreType.DMA,
        pltpu.SemaphoreType.DMA,
        pltpu.SemaphoreType.DMA,
    ],
)
def _scatter_kernel(adj_hbm, xs_hbm, out_hbm,
                    ridx_v, cidx0_v, cidx1_v, cidx2_v,
                    rows0_v, rows1_v, rows2_v, acc_sh,
                    gsem0, gsem1, gsem2, csem0, csem1, csem2):
    cid = lax.axis_index("c")
    sid = lax.axis_index("s")
    tile = cid * NTILE + sid

    _zero_vmem_2d(rows0_v, CH, F)
    pltpu.sync_copy(adj_hbm.at[0, tile], ridx_v)
    for j in range(RPT // CH):
        pltpu.sync_copy(rows0_v, acc_sh.at[pl.ds(sid * RPT + j * CH, CH)])
    plsc.subcore_barrier()

    rbufs = (rows0_v, rows1_v, rows2_v)
    gsems = (gsem0, gsem1, gsem2)
    cbufs = (cidx0_v, cidx1_v, cidx2_v)
    csems = (csem0, csem1, csem2)

    def start_gather(i, b):
        pltpu.async_copy(xs_hbm.at[ridx_v.at[i]], rbufs[b], gsems[b])

    def wait_gather(i, b):
        pltpu.make_async_copy(xs_hbm.at[ridx_v.at[i]], rbufs[b], gsems[b]).wait()

    def start_cidx(i, b):
        pltpu.async_copy(adj_hbm.at[1, tile, i], cbufs[b], csems[b])

    def wait_cidx(i, b):
        pltpu.make_async_copy(adj_hbm.at[1, tile, i], cbufs[b], csems[b]).wait()

    start_cidx(0, 0)
    start_cidx(1, 1)
    start_gather(0, 0)
    start_gather(1, 1)

    def body(g, _):
        for b in range(3):
            i = g * 3 + b
            b2 = (b + 2) % 3

            @pl.when(i < NCHUNK)
            def _():
                wait_gather(i, b)

                @pl.when(i + 2 < NCHUNK)
                def _():
                    start_gather(i + 2, b2)
                    start_cidx(i + 2, b2)

                wait_cidx(i, b)
                pltpu.sync_copy(rbufs[b], acc_sh.at[cbufs[b]], add=True)
        return 0

    lax.fori_loop(0, (NCHUNK + 2) // 3, body, 0)
    plsc.subcore_barrier()
    pltpu.sync_copy(acc_sh.at[pl.ds(sid * RPT, RPT)],
                    out_hbm.at[cid, pl.ds(sid * RPT, RPT)])


def _final_body(acc_ref, deg_ref, x_ref, ew_ref, w1_ref, w2_ref, out_ref):
    a = acc_ref[0] + acc_ref[1]
    d = deg_ref[0] + deg_ref[1]
    dis = jnp.where(d > 0.0, lax.rsqrt(jnp.maximum(d, 1e-30)), 0.0)
    xv = x_ref[...]
    A = jnp.dot(a, w1_ref[...], preferred_element_type=jnp.float32)
    Bm = jnp.dot(xv, w2_ref[...], preferred_element_type=jnp.float32)
    o = xv
    for e in range(NENV):
        we = ew_ref[:, e:e + 1]
        o = o + (we * dis) * A[:, F * e:F * (e + 1)]
        o = o + we * Bm[:, F * e:F * (e + 1)]
    out_ref[...] = o


def _final(acc2, deg2, x, ew, w1, w2):
    blk = 2000
    grid = N // blk
    return pl.pallas_call(
        _final_body,
        grid=(grid,),
        in_specs=[
            pl.BlockSpec((NSC, blk, F), lambda i: (0, i, 0)),
            pl.BlockSpec((NSC, blk, 1), lambda i: (0, i, 0)),
            pl.BlockSpec((blk, F), lambda i: (i, 0)),
            pl.BlockSpec((blk, NENV), lambda i: (i, 0)),
            pl.BlockSpec((F, NENV * F), lambda i: (0, 0)),
            pl.BlockSpec((F, NENV * F), lambda i: (0, 0)),
        ],
        out_specs=pl.BlockSpec((blk, F), lambda i: (i, 0)),
        out_shape=jax.ShapeDtypeStruct((N, F), jnp.float32),
    )(acc2, deg2, x, ew, w1, w2)


def kernel(x, adj, env_weights, weights):
    adjr = adj.astype(jnp.int32).reshape(2, NSC * NTILE, NCHUNK, CH)

    deg2 = _deg_kernel(adjr)
    deg3 = deg2.reshape(NSC, NP, 1)
    xs = _prep(deg3, x)
    acc2 = _scatter_kernel(adjr, xs)

    w1 = jnp.transpose(weights[:, :F, :], (1, 0, 2)).reshape(F, NENV * F)
    w2 = jnp.transpose(weights[:, F:, :], (1, 0, 2)).reshape(F, NENV * F)
    return _final(acc2, deg3, x, env_weights, w1, w2)


# speedup vs baseline: 2.0408x; 1.0096x over previous
import functools

import jax
import jax.numpy as jnp
from jax import lax
from jax.experimental import pallas as pl
from jax.experimental.pallas import tpu as pltpu
from jax.experimental.pallas import tpu_sc as plsc

N = 10000
E = 320000
F = 128
NENV = 4
NP = 10240
NSC = 2
NTILE = 16
TPE = E // (NSC * NTILE)
CH = 80
NCHUNK = TPE // CH
RPT = NP // NTILE

_MESH = plsc.VectorSubcoreMesh(core_axis_name="c", subcore_axis_name="s")


def _zero_vmem_2d(ref, nrows, ncols):
    z = jnp.zeros((16,), jnp.float32)
    cpr = ncols // 16

    def body(i, _):
        r = i // cpr
        c = i % cpr
        ref[r, pl.ds(c * 16, 16)] = z
        return 0

    lax.fori_loop(0, nrows * cpr, body, 0)


def _zero_vmem_1d(ref, n):
    z = jnp.zeros((16,), jnp.float32)

    def body(i, _):
        ref[pl.ds(i * 16, 16)] = z
        return 0

    lax.fori_loop(0, n // 16, body, 0)


@functools.partial(
    pl.kernel,
    out_type=jax.ShapeDtypeStruct((NSC, NP), jnp.float32),
    mesh=_MESH,
    scratch_types=[
        pltpu.VMEM((NCHUNK, CH), jnp.int32),
        pltpu.VMEM((CH,), jnp.float32),
        pltpu.VMEM((RPT,), jnp.float32),
        pltpu.VMEM_SHARED((NP,), jnp.float32),
        pltpu.SemaphoreType.DMA,
    ],
)
def _deg_kernel(adj_hbm, out_hbm, cidx_v, ones_v, zbuf_v, deg_sh, sem):
    cid = lax.axis_index("c")
    sid = lax.axis_index("s")
    tile = cid * NTILE + sid

    _zero_vmem_1d(zbuf_v, RPT)
    o = jnp.ones((16,), jnp.float32)
    for i in range(CH // 16):
        ones_v[pl.ds(i * 16, 16)] = o
    pltpu.sync_copy(adj_hbm.at[1, tile], cidx_v)
    pltpu.sync_copy(zbuf_v, deg_sh.at[pl.ds(sid * RPT, RPT)])
    plsc.subcore_barrier()

    def fire(i, _):
        pltpu.async_copy(ones_v, deg_sh.at[cidx_v.at[i]], sem, add=True)
        return 0

    lax.fori_loop(0, NCHUNK, fire, 0)

    def drain(i, _):
        pltpu.make_async_copy(ones_v, deg_sh.at[cidx_v.at[0]], sem).wait()
        return 0

    lax.fori_loop(0, NCHUNK, drain, 0)
    plsc.subcore_barrier()
    pltpu.sync_copy(deg_sh.at[pl.ds(sid * RPT, RPT)],
                    out_hbm.at[cid, pl.ds(sid * RPT, RPT)])


def _prep_body(deg_ref, x_ref, xs_ref):
    d = deg_ref[0] + deg_ref[1]
    dis = jnp.where(d > 0.0, lax.rsqrt(jnp.maximum(d, 1e-30)), 0.0)
    xs_ref[...] = x_ref[...] * dis


def _prep(deg2, x):
    blk = 2000
    grid = N // blk
    return pl.pallas_call(
        _prep_body,
        grid=(grid,),
        in_specs=[
            pl.BlockSpec((NSC, blk, 1), lambda i: (0, i, 0)),
            pl.BlockSpec((blk, F), lambda i: (i, 0)),
        ],
        out_specs=pl.BlockSpec((blk, F), lambda i: (i, 0)),
        out_shape=jax.ShapeDtypeStruct((N, F), jnp.float32),
    )(deg2, x)


@functools.partial(
    pl.kernel,
    out_type=jax.ShapeDtypeStruct((NSC, NP, F), jnp.float32),
    mesh=_MESH,
    scratch_types=[
        pltpu.VMEM((NCHUNK, CH), jnp.int32),
        pltpu.VMEM((CH,), jnp.int32),
        pltpu.VMEM((CH,), jnp.int32),
        pltpu.VMEM((CH,), jnp.int32),
        pltpu.VMEM((CH, F), jnp.float32),
        pltpu.VMEM((CH, F), jnp.float32),
        pltpu.VMEM((CH, F), jnp.float32),
        pltpu.VMEM_SHARED((NP, F), jnp.float32),
        pltpu.SemaphoreType.DMA,
        pltpu.SemaphoreType.DMA,
        pltpu.SemaphoreType.DMA,
        pltpu.SemaphoreType.DMA,
        pltpu.SemaphoreType.DMA,
        pltpu.SemaphoreType.DMA,
    ],
)
def _scatter_kernel(adj_hbm, xs_hbm, out_hbm,
                    ridx_v, cidx0_v, cidx1_v, cidx2_v,
                    rows0_v, rows1_v, rows2_v, acc_sh,
                    gsem0, gsem1, gsem2, csem0, csem1, csem2):
    cid = lax.axis_index("c")
    sid = lax.axis_index("s")
    tile = cid * NTILE + sid

    _zero_vmem_2d(rows0_v, CH, F)
    pltpu.sync_copy(adj_hbm.at[0, tile], ridx_v)
    for j in range(RPT // CH):
        pltpu.sync_copy(rows0_v, acc_sh.at[pl.ds(sid * RPT + j * CH, CH)])
    plsc.subcore_barrier()

    rbufs = (rows0_v, rows1_v, rows2_v)
    gsems = (gsem0, gsem1, gsem2)
    cbufs = (cidx0_v, cidx1_v, cidx2_v)
    csems = (csem0, csem1, csem2)

    def start_gather(i, b):
        pltpu.async_copy(xs_hbm.at[ridx_v.at[i]], rbufs[b], gsems[b])

    def wait_gather(i, b):
        pltpu.make_async_copy(xs_hbm.at[ridx_v.at[i]], rbufs[b], gsems[b]).wait()

    def start_cidx(i, b):
        pltpu.async_copy(adj_hbm.at[1, tile, i], cbufs[b], csems[b])

    def wait_cidx(i, b):
        pltpu.make_async_copy(adj_hbm.at[1, tile, i], cbufs[b], csems[b]).wait()

    start_cidx(0, 0)
    start_cidx(1, 1)
    start_gather(0, 0)
    start_gather(1, 1)

    def body(g, _):
        for b in range(3):
            i = g * 3 + b
            b2 = (b + 2) % 3

            @pl.when(i < NCHUNK)
            def _():
                wait_gather(i, b)

                @pl.when(i + 2 < NCHUNK)
                def _():
                    start_gather(i + 2, b2)
                    start_cidx(i + 2, b2)

                wait_cidx(i, b)
                pltpu.sync_copy(rbufs[b], acc_sh.at[cbufs[b]], add=True)
        return 0

    lax.fori_loop(0, (NCHUNK + 2) // 3, body, 0)
    plsc.subcore_barrier()
    pltpu.sync_copy(acc_sh.at[pl.ds(sid * RPT, RPT)],
                    out_hbm.at[cid, pl.ds(sid * RPT, RPT)])


def _xpart_body(x_ref, ew_ref, w2_ref, out_ref):
    xv = x_ref[...]
    Bm = jnp.dot(xv, w2_ref[...], preferred_element_type=jnp.float32)
    o = xv
    for e in range(NENV):
        o = o + ew_ref[:, e:e + 1] * Bm[:, F * e:F * (e + 1)]
    out_ref[...] = o


def _xpart(x, ew, w2):
    blk = 2000
    grid = N // blk
    return pl.pallas_call(
        _xpart_body,
        grid=(grid,),
        in_specs=[
            pl.BlockSpec((blk, F), lambda i: (i, 0)),
            pl.BlockSpec((blk, NENV), lambda i: (i, 0)),
            pl.BlockSpec((F, NENV * F), lambda i: (0, 0)),
        ],
        out_specs=pl.BlockSpec((blk, F), lambda i: (i, 0)),
        out_shape=jax.ShapeDtypeStruct((N, F), jnp.float32),
    )(x, ew, w2)


def _final_body(acc_ref, deg_ref, o0_ref, ew_ref, w1_ref, out_ref):
    a = acc_ref[0] + acc_ref[1]
    d = deg_ref[0] + deg_ref[1]
    dis = jnp.where(d > 0.0, lax.rsqrt(jnp.maximum(d, 1e-30)), 0.0)
    A = jnp.dot(a, w1_ref[...], preferred_element_type=jnp.float32)
    o = o0_ref[...]
    for e in range(NENV):
        o = o + (ew_ref[:, e:e + 1] * dis) * A[:, F * e:F * (e + 1)]
    out_ref[...] = o


def _final(acc2, deg2, o0, ew, w1):
    blk = 2000
    grid = N // blk
    return pl.pallas_call(
        _final_body,
        grid=(grid,),
        in_specs=[
            pl.BlockSpec((NSC, blk, F), lambda i: (0, i, 0)),
            pl.BlockSpec((NSC, blk, 1), lambda i: (0, i, 0)),
            pl.BlockSpec((blk, F), lambda i: (i, 0)),
            pl.BlockSpec((blk, NENV), lambda i: (i, 0)),
            pl.BlockSpec((F, NENV * F), lambda i: (0, 0)),
        ],
        out_specs=pl.BlockSpec((blk, F), lambda i: (i, 0)),
        out_shape=jax.ShapeDtypeStruct((N, F), jnp.float32),
    )(acc2, deg2, o0, ew, w1)


def kernel(x, adj, env_weights, weights):
    adjr = adj.astype(jnp.int32).reshape(2, NSC * NTILE, NCHUNK, CH)

    deg2 = _deg_kernel(adjr)
    deg3 = deg2.reshape(NSC, NP, 1)
    xs = _prep(deg3, x)
    acc2 = _scatter_kernel(adjr, xs)

    w1 = jnp.transpose(weights[:, :F, :], (1, 0, 2)).reshape(F, NENV * F)
    w2 = jnp.transpose(weights[:, F:, :], (1, 0, 2)).reshape(F, NENV * F)
    o0 = _xpart(x, env_weights, w2)
    return _final(acc2, deg3, o0, env_weights, w1)
